# Initial kernel scaffold; baseline (speedup 1.0000x reference)
import jax
import jax.numpy as jnp
from jax.experimental import pallas as pl


def kernel(bboxes, cls_scores, bbox_preds, gt_bboxes, bbox_levels, gt_labels):
    raise NotImplementedError("write your pallas kernel here")


# speedup vs baseline: 1.7413x; 1.7413x over previous
import functools

import jax
import jax.numpy as jnp
from jax.experimental import pallas as pl
from jax.experimental.pallas import tpu as pltpu
from jax.experimental.pallas import tpu_sc as plsc

EPS = 1e-10
ALPHA = 0.8
TOPK = 9
NPAD = 20480
CAND = 960
TRASH = 20000


def _tc_body(preds_ref, bx_ref, cls_ref, gtt_ref, lab_ref, nx_ref, ny_ref,
             idx_ref, w_ref):
    N = preds_ref.shape[0]
    C = cls_ref.shape[1]
    G = gtt_ref.shape[1]
    B = 2000
    NB = N // B

    gx1 = gtt_ref[0:1, :]
    gy1 = gtt_ref[1:2, :]
    gx2 = gtt_ref[2:3, :]
    gy2 = gtt_ref[3:4, :]
    area2 = (gx2 - gx1) * (gy2 - gy1)

    labels = lab_ref[0:1, :]
    cls_ids = jax.lax.broadcasted_iota(jnp.int32, (C, G), 0)
    onehot = (cls_ids == jnp.broadcast_to(labels, (C, G))).astype(jnp.float32)

    PAD = 16

    def block_step(i, carry):
        vals, idxs, cxs, cys = carry
        base = i * B
        pb = preds_ref[pl.ds(base, B), :]
        px1 = pb[:, 0:1]
        py1 = pb[:, 1:2]
        px2 = pb[:, 2:3]
        py2 = pb[:, 3:4]
        area1 = (px2 - px1) * (py2 - py1)
        iw = jnp.maximum(jnp.minimum(px2, gx2) - jnp.maximum(px1, gx1), 0.0)
        ih = jnp.maximum(jnp.minimum(py2, gy2) - jnp.maximum(py1, gy1), 0.0)
        inter = iw * ih
        union = jnp.maximum(area1 + area2 - inter, 1e-6)
        iou = inter / union
        ov_pow = jnp.where(iou > 0.0, jnp.maximum(iou, EPS) ** ALPHA, 0.0)

        cb = cls_ref[pl.ds(base, B), :]
        cls_pow = jax.nn.sigmoid(cb) ** (1.0 - ALPHA)
        cls_sel = jnp.dot(
            cls_pow,
            onehot,
            preferred_element_type=jnp.float32,
            precision=jax.lax.Precision.HIGHEST,
        )
        score = cls_sel * ov_pow

        bb = bx_ref[pl.ds(base, B), :]
        bcx = (bb[:, 0:1] + bb[:, 2:3]) * 0.5
        bcy = (bb[:, 1:2] + bb[:, 3:4]) * 0.5

        rid = jax.lax.broadcasted_iota(jnp.int32, (B, G), 0) + base
        pool_v = jnp.concatenate([score, vals], axis=0)
        pool_i = jnp.concatenate([rid, idxs], axis=0)
        pool_x = jnp.concatenate([jnp.broadcast_to(bcx, (B, G)), cxs], axis=0)
        pool_y = jnp.concatenate([jnp.broadcast_to(bcy, (B, G)), cys], axis=0)

        vr, ir, xr, yr = [], [], [], []
        for _ in range(TOPK):
            m = jnp.max(pool_v, axis=0, keepdims=True)
            ismax = pool_v == m
            sel = jnp.min(
                jnp.where(ismax, pool_i, jnp.int32(2**31 - 1)),
                axis=0,
                keepdims=True,
            )
            issel = ismax & (pool_i == sel)
            vr.append(m)
            ir.append(sel)
            xr.append(jnp.sum(jnp.where(issel, pool_x, 0.0), axis=0, keepdims=True))
            yr.append(jnp.sum(jnp.where(issel, pool_y, 0.0), axis=0, keepdims=True))
            pool_v = jnp.where(issel, -1.0, pool_v)

        pad_rows = PAD - TOPK
        vr.append(jnp.full((pad_rows, G), -1.0, jnp.float32))
        ir.append(jnp.full((pad_rows, G), 2**30, jnp.int32))
        xr.append(jnp.zeros((pad_rows, G), jnp.float32))
        yr.append(jnp.zeros((pad_rows, G), jnp.float32))
        return (
            jnp.concatenate(vr, axis=0),
            jnp.concatenate(ir, axis=0),
            jnp.concatenate(xr, axis=0),
            jnp.concatenate(yr, axis=0),
        )

    init = (
        jnp.full((PAD, G), -1.0, jnp.float32),
        jnp.full((PAD, G), 2**30, jnp.int32),
        jnp.zeros((PAD, G), jnp.float32),
        jnp.zeros((PAD, G), jnp.float32),
    )
    vals, idxs, cxs, cys = jax.lax.fori_loop(0, NB, block_step, init)

    cidx = idxs[:TOPK]
    ccx = cxs[:TOPK]
    ccy = cys[:TOPK]

    dx = ccx + nx_ref[:TOPK]
    dy = ccy + ny_ref[:TOPK]
    mx = jnp.mean(dx, axis=0, keepdims=True)
    my = jnp.mean(dy, axis=0, keepdims=True)
    ddx = dx - mx
    ddy = dy - my
    sxx = jnp.mean(ddx * ddx, axis=0, keepdims=True)
    sxy = jnp.mean(ddx * ddy, axis=0, keepdims=True)
    syy = jnp.mean(ddy * ddy, axis=0, keepdims=True)
    denom = sxx * syy - sxy * sxy + 1e-10
    i00 = syy / denom
    i01 = -sxy / denom
    i11 = sxx / denom
    pxd = ccx - mx
    pyd = ccy - my
    q = i00 * pxd * pxd + 2.0 * i01 * pxd * pyd + i11 * pyd * pyd
    cw = jnp.exp(-0.5 * q)

    valid = (
        (ccx - gx1 > EPS)
        & (ccy - gy1 > EPS)
        & (gx2 - ccx > EPS)
        & (gy2 - ccy > EPS)
    )
    wv = jnp.maximum(jnp.where(valid, cw, 0.0), 0.0)

    rid2 = (
        jax.lax.broadcasted_iota(jnp.int32, (TOPK, G), 0) * 128
        + jax.lax.broadcasted_iota(jnp.int32, (TOPK, G), 1)
    )
    cidx3 = cidx[:, :, None]
    best = jnp.full((TOPK, G), -1.0, jnp.float32)
    for k in range(TOPK):
        ck = cidx[k : k + 1, :][None]
        wk = wv[k : k + 1, :][None]
        eq3 = cidx3 == ck
        best = jnp.maximum(best, jnp.max(jnp.where(eq3, wk, -1.0), axis=2))
    bestrid = jnp.full((TOPK, G), 2**30, jnp.int32)
    for k in range(TOPK):
        ck = cidx[k : k + 1, :][None]
        wk = wv[k : k + 1, :][None]
        rk = rid2[k : k + 1, :][None]
        cond = (cidx3 == ck) & (wk == best[:, :, None])
        bestrid = jnp.minimum(
            bestrid, jnp.min(jnp.where(cond, rk, 2**30), axis=2)
        )
    keep = rid2 == bestrid
    idx_final = jnp.where(keep, cidx, TRASH)
    w_final = jnp.where(keep, wv, 0.0)

    pad_rows = 16 - TOPK
    idx_ref[...] = jnp.concatenate(
        [idx_final, jnp.full((pad_rows, G), TRASH, jnp.int32)], axis=0
    )
    w_ref[...] = jnp.concatenate(
        [w_final, jnp.zeros((pad_rows, G), jnp.float32)], axis=0
    )


def _sc_body(idx_hbm, w_hbm, out_hbm, idx_v, w_v, out_v):
    nw = 32
    per = NPAD // nw
    wid = jax.lax.axis_index("s") * 2 + jax.lax.axis_index("c")
    base = wid * per

    pltpu.sync_copy(idx_hbm, idx_v)
    pltpu.sync_copy(w_hbm, w_v)

    zeros16 = jnp.zeros((16,), jnp.float32)
    for i in range(per // 16):
        out_v[pl.ds(i * 16, 16)] = zeros16

    for j in range(CAND // 16):
        iv = idx_v[pl.ds(j * 16, 16)]
        wv_ = w_v[pl.ds(j * 16, 16)]
        msk = (iv >= base) & (iv < base + per)
        loc = jnp.where(msk, iv - base, 0)
        plsc.store_scatter(out_v, [loc], wv_, mask=msk)

    pltpu.sync_copy(out_v, out_hbm.at[pl.ds(base, per)])


def kernel(bboxes, cls_scores, bbox_preds, gt_bboxes, bbox_levels, gt_labels):
    del bbox_levels
    N, C = cls_scores.shape
    G = gt_bboxes.shape[0]

    noise = (
        jax.random.uniform(jax.random.key(1), (G, TOPK, 2), dtype=jnp.float32) - 0.5
    ) * 0.1
    nx = noise[:, :, 0].T
    ny = noise[:, :, 1].T

    gt_t = gt_bboxes[:, :4].T.astype(jnp.float32)
    labels2 = gt_labels.astype(jnp.int32).reshape(1, G)

    idx_o, w_o = pl.pallas_call(
        _tc_body,
        out_shape=(
            jax.ShapeDtypeStruct((16, G), jnp.int32),
            jax.ShapeDtypeStruct((16, G), jnp.float32),
        ),
    )(
        bbox_preds.astype(jnp.float32),
        bboxes[:, :4].astype(jnp.float32),
        cls_scores.astype(jnp.float32),
        gt_t,
        labels2,
        nx,
        ny,
    )

    pad = CAND - TOPK * G
    idx_flat = jnp.concatenate(
        [idx_o[:TOPK].reshape(-1), jnp.full((pad,), TRASH, jnp.int32)]
    )
    w_flat = jnp.concatenate(
        [w_o[:TOPK].reshape(-1), jnp.zeros((pad,), jnp.float32)]
    )

    mesh = plsc.VectorSubcoreMesh(core_axis_name="c", subcore_axis_name="s")
    sc = functools.partial(
        pl.kernel,
        mesh=mesh,
        compiler_params=pltpu.CompilerParams(needs_layout_passes=False),
        out_type=jax.ShapeDtypeStruct((NPAD,), jnp.float32),
        scratch_types=[
            pltpu.VMEM((CAND,), jnp.int32),
            pltpu.VMEM((CAND,), jnp.float32),
            pltpu.VMEM((NPAD // 32,), jnp.float32),
        ],
    )(_sc_body)
    out_pad = sc(idx_flat, w_flat)
    return out_pad[:N]


# speedup vs baseline: 1.8682x; 1.0729x over previous
import functools

import jax
import jax.numpy as jnp
from jax.experimental import pallas as pl
from jax.experimental.pallas import tpu as pltpu
from jax.experimental.pallas import tpu_sc as plsc

EPS = 1e-10
ALPHA = 0.8
TOPK = 9
NPAD = 20480
GPAD = 112
CAND = TOPK * GPAD
TRASH = 20000


def _tc_body(preds_ref, bx_ref, cls_ref, gtt_ref, lab_ref, idx_ref):
    N = preds_ref.shape[0]
    C = cls_ref.shape[1]
    G = gtt_ref.shape[1]
    B = 2000
    NB = N // B

    gx1 = gtt_ref[0:1, :]
    gy1 = gtt_ref[1:2, :]
    gx2 = gtt_ref[2:3, :]
    gy2 = gtt_ref[3:4, :]
    area2 = (gx2 - gx1) * (gy2 - gy1)
    del bx_ref

    labels = lab_ref[0:1, :]
    cls_ids = jax.lax.broadcasted_iota(jnp.int32, (C, G), 0)
    onehot = (cls_ids == jnp.broadcast_to(labels, (C, G))).astype(jnp.float32)

    PAD = 16

    def block_step(i, carry):
        vals, idxs = carry
        base = i * B
        pb = preds_ref[pl.ds(base, B), :]
        px1 = pb[:, 0:1]
        py1 = pb[:, 1:2]
        px2 = pb[:, 2:3]
        py2 = pb[:, 3:4]
        area1 = (px2 - px1) * (py2 - py1)
        iw = jnp.maximum(jnp.minimum(px2, gx2) - jnp.maximum(px1, gx1), 0.0)
        ih = jnp.maximum(jnp.minimum(py2, gy2) - jnp.maximum(py1, gy1), 0.0)
        inter = iw * ih
        union = jnp.maximum(area1 + area2 - inter, 1e-6)
        iou = inter / union
        ov_pow = jnp.where(iou > 0.0, jnp.maximum(iou, EPS) ** ALPHA, 0.0)

        cb = cls_ref[pl.ds(base, B), :]
        cls_pow = jax.nn.sigmoid(cb) ** (1.0 - ALPHA)
        cls_sel = jnp.dot(
            cls_pow,
            onehot,
            preferred_element_type=jnp.float32,
            precision=jax.lax.Precision.HIGHEST,
        )
        score = cls_sel * ov_pow

        rid = jax.lax.broadcasted_iota(jnp.int32, (B, G), 0) + base
        pool_v = jnp.concatenate([score, vals], axis=0)
        pool_i = jnp.concatenate([rid, idxs], axis=0)

        vr, ir = [], []
        for _ in range(TOPK):
            m = jnp.max(pool_v, axis=0, keepdims=True)
            ismax = pool_v == m
            sel = jnp.min(
                jnp.where(ismax, pool_i, jnp.int32(2**31 - 1)),
                axis=0,
                keepdims=True,
            )
            issel = ismax & (pool_i == sel)
            vr.append(m)
            ir.append(sel)
            pool_v = jnp.where(issel, -1.0, pool_v)

        pad_rows = PAD - TOPK
        vr.append(jnp.full((pad_rows, G), -1.0, jnp.float32))
        ir.append(jnp.full((pad_rows, G), 2**30, jnp.int32))
        return jnp.concatenate(vr, axis=0), jnp.concatenate(ir, axis=0)

    init = (
        jnp.full((PAD, G), -1.0, jnp.float32),
        jnp.full((PAD, G), 2**30, jnp.int32),
    )
    _, idxs = jax.lax.fori_loop(0, NB, block_step, init)
    idx_ref[...] = idxs


def _sc_body(
    idx_hbm, nx_hbm, ny_hbm, g1_hbm, g2_hbm, g3_hbm, g4_hbm, bflat_hbm,
    out_hbm,
    idx_v, iva, ivb, ivc, ivd, xa, xb, ya, yb, nx_v, ny_v,
    g1_v, g2_v, g3_v, g4_v, w_v, tmp_i, tmp_w, out_v, w_sh, sem,
):
    n_real = NPAD - 480
    cid = jax.lax.axis_index("c")
    sid = jax.lax.axis_index("s")
    wid = sid * 2 + cid
    per = NPAD // 32
    base = wid * per

    pltpu.sync_copy(idx_hbm, idx_v)

    @pl.when(sid == 0)
    def _gauss():
        pltpu.sync_copy(nx_hbm, nx_v)
        pltpu.sync_copy(ny_hbm, ny_v)
        pltpu.sync_copy(g1_hbm, g1_v)
        pltpu.sync_copy(g2_hbm, g2_v)
        pltpu.sync_copy(g3_hbm, g3_v)
        pltpu.sync_copy(g4_hbm, g4_v)

        def scale(j, _):
            iv = jnp.minimum(idx_v[pl.ds(j * 16, 16)], n_real - 1) * 4
            iva[pl.ds(j * 16, 16)] = iv
            ivb[pl.ds(j * 16, 16)] = iv + 2
            ivc[pl.ds(j * 16, 16)] = iv + 1
            ivd[pl.ds(j * 16, 16)] = iv + 3
            return 0

        jax.lax.fori_loop(0, CAND // 16, scale, 0)

        cps = []
        for ivref, dst in ((iva, xa), (ivb, xb), (ivc, ya), (ivd, yb)):
            for c in range(TOPK):
                cp = pltpu.make_async_copy(
                    bflat_hbm.at[ivref.at[pl.ds(c * GPAD, GPAD)]],
                    dst.at[pl.ds(c * GPAD, GPAD)],
                    sem,
                )
                cp.start()
                cps.append(cp)
        for cp in cps:
            cp.wait()

        glane = jax.lax.iota(jnp.int32, 16)
        for ch in range(GPAD // 16):
            s = ch * 16
            laneok = (glane + s) < 100
            c1 = g1_v[pl.ds(s, 16)]
            c2 = g2_v[pl.ds(s, 16)]
            c3 = g3_v[pl.ds(s, 16)]
            c4 = g4_v[pl.ds(s, 16)]
            cxs, cys, nxs, nys = [], [], [], []
            sx = jnp.zeros((16,), jnp.float32)
            sy = jnp.zeros((16,), jnp.float32)
            for k in range(TOPK):
                o = k * GPAD + s
                cx = (xa[pl.ds(o, 16)] + xb[pl.ds(o, 16)]) * 0.5
                cy = (ya[pl.ds(o, 16)] + yb[pl.ds(o, 16)]) * 0.5
                nk = nx_v[pl.ds(o, 16)]
                mk = ny_v[pl.ds(o, 16)]
                cxs.append(cx)
                cys.append(cy)
                nxs.append(nk)
                nys.append(mk)
                sx = sx + (cx + nk)
                sy = sy + (cy + mk)
            mx = sx / 9.0
            my = sy / 9.0
            sxx = jnp.zeros((16,), jnp.float32)
            sxy = jnp.zeros((16,), jnp.float32)
            syy = jnp.zeros((16,), jnp.float32)
            for k in range(TOPK):
                dxn = (cxs[k] + nxs[k]) - mx
                dyn = (cys[k] + nys[k]) - my
                sxx = sxx + dxn * dxn
                sxy = sxy + dxn * dyn
                syy = syy + dyn * dyn
            sxx = sxx / 9.0
            sxy = sxy / 9.0
            syy = syy / 9.0
            den = sxx * syy - sxy * sxy + 1e-10
            i00 = syy / den
            i01 = -sxy / den
            i11 = sxx / den
            for k in range(TOPK):
                px = cxs[k] - mx
                py = cys[k] - my
                q = i00 * px * px + 2.0 * i01 * px * py + i11 * py * py
                w = jnp.exp(-0.5 * q)
                valid = (
                    (cxs[k] - c1 > EPS)
                    & (cys[k] - c2 > EPS)
                    & (c3 - cxs[k] > EPS)
                    & (c4 - cys[k] > EPS)
                )
                w = jnp.where(valid & laneok, w, 0.0)
                w_v[pl.ds(k * GPAD + s, 16)] = w

        rot_vecs = [
            (jax.lax.iota(jnp.int32, 16) + r) & 15 for r in range(1, 16)
        ]

        def dedup(j, _):
            iv = idx_v[pl.ds(j * 16, 16)]
            wv = w_v[pl.ds(j * 16, 16)]
            tmp_i[...] = iv
            tmp_w[...] = wv
            acc = wv
            for rv in rot_vecs:
                ir = plsc.load_gather(tmp_i, [rv])
                wr = plsc.load_gather(tmp_w, [rv])
                acc = jnp.where(ir == iv, jnp.maximum(acc, wr), acc)
            w_v[pl.ds(j * 16, 16)] = acc
            return 0

        jax.lax.fori_loop(0, CAND // 16, dedup, 0)
        pltpu.sync_copy(w_v, w_sh)

    plsc.subcore_barrier()
    pltpu.sync_copy(w_sh, w_v)

    zeros16 = jnp.zeros((16,), jnp.float32)

    def zero(i, _):
        out_v[pl.ds(i * 16, 16)] = zeros16
        return 0

    jax.lax.fori_loop(0, per // 16, zero, 0)

    def scat(j, _):
        iv = idx_v[pl.ds(j * 16, 16)]
        wv = w_v[pl.ds(j * 16, 16)]
        msk = (iv >= base) & (iv < base + per)
        loc = jnp.where(msk, iv - base, 0)
        old = plsc.load_gather(out_v, [loc], mask=msk)
        new = jnp.where(msk, jnp.maximum(old, wv), wv)
        plsc.store_scatter(out_v, [loc], new, mask=msk)
        return 0

    jax.lax.fori_loop(0, CAND // 16, scat, 0)
    pltpu.sync_copy(out_v, out_hbm.at[pl.ds(base, per)])


def kernel(bboxes, cls_scores, bbox_preds, gt_bboxes, bbox_levels, gt_labels):
    del bbox_levels
    N, C = cls_scores.shape
    G = gt_bboxes.shape[0]

    gt_t = gt_bboxes[:, :4].T.astype(jnp.float32)
    labels2 = gt_labels.astype(jnp.int32).reshape(1, G)

    idx_o = pl.pallas_call(
        _tc_body,
        out_shape=jax.ShapeDtypeStruct((16, G), jnp.int32),
    )(
        bbox_preds.astype(jnp.float32),
        bboxes[:, :4].astype(jnp.float32),
        cls_scores.astype(jnp.float32),
        gt_t,
        labels2,
    )

    noise = (
        jax.random.uniform(jax.random.key(1), (G, TOPK, 2), dtype=jnp.float32) - 0.5
    ) * 0.1
    nxp = jnp.zeros((TOPK, GPAD), jnp.float32).at[:, :G].set(noise[:, :, 0].T)
    nyp = jnp.zeros((TOPK, GPAD), jnp.float32).at[:, :G].set(noise[:, :, 1].T)
    idxp = (
        jnp.full((TOPK, GPAD), TRASH, jnp.int32)
        .at[:, :G]
        .set(idx_o[:TOPK])
        .reshape(-1)
    )
    gpad = jnp.zeros((4, GPAD), jnp.float32).at[:, :G].set(gt_t)
    bflat = bboxes[:, :4].astype(jnp.float32).reshape(-1)

    mesh = plsc.VectorSubcoreMesh(core_axis_name="c", subcore_axis_name="s")
    sc = functools.partial(
        pl.kernel,
        mesh=mesh,
        compiler_params=pltpu.CompilerParams(needs_layout_passes=False),
        out_type=jax.ShapeDtypeStruct((NPAD,), jnp.float32),
        scratch_types=[
            pltpu.VMEM((CAND,), jnp.int32),
            pltpu.VMEM((CAND,), jnp.int32),
            pltpu.VMEM((CAND,), jnp.int32),
            pltpu.VMEM((CAND,), jnp.int32),
            pltpu.VMEM((CAND,), jnp.int32),
            pltpu.VMEM((CAND,), jnp.float32),
            pltpu.VMEM((CAND,), jnp.float32),
            pltpu.VMEM((CAND,), jnp.float32),
            pltpu.VMEM((CAND,), jnp.float32),
            pltpu.VMEM((CAND,), jnp.float32),
            pltpu.VMEM((CAND,), jnp.float32),
            pltpu.VMEM((GPAD,), jnp.float32),
            pltpu.VMEM((GPAD,), jnp.float32),
            pltpu.VMEM((GPAD,), jnp.float32),
            pltpu.VMEM((GPAD,), jnp.float32),
            pltpu.VMEM((CAND,), jnp.float32),
            pltpu.VMEM((16,), jnp.int32),
            pltpu.VMEM((16,), jnp.float32),
            pltpu.VMEM((NPAD // 32,), jnp.float32),
            pltpu.VMEM_SHARED((CAND,), jnp.float32),
            pltpu.SemaphoreType.DMA,
        ],
    )(_sc_body)
    out_pad = sc(
        idxp,
        nxp.reshape(-1),
        nyp.reshape(-1),
        gpad[0],
        gpad[1],
        gpad[2],
        gpad[3],
        bflat,
    )
    return out_pad[:N]


# speedup vs baseline: 2.0412x; 1.0926x over previous
import functools

import jax
import jax.numpy as jnp
from jax.experimental import pallas as pl
from jax.experimental.pallas import tpu as pltpu
from jax.experimental.pallas import tpu_sc as plsc

EPS = 1e-10
ALPHA = 0.8
TOPK = 9
NPAD = 20480
GPAD = 112
CAND = TOPK * GPAD
TRASH = 20000


def _tc_body(preds_ref, cls_ref, gtt_ref, lab_ref, idx_ref):
    N = preds_ref.shape[0]
    C = cls_ref.shape[1]
    G = gtt_ref.shape[1]
    B = 4000
    NB = N // B

    gx1 = gtt_ref[0:1, :]
    gy1 = gtt_ref[1:2, :]
    gx2 = gtt_ref[2:3, :]
    gy2 = gtt_ref[3:4, :]
    area2 = (gx2 - gx1) * (gy2 - gy1)

    labels = lab_ref[0:1, :]
    cls_ids = jax.lax.broadcasted_iota(jnp.int32, (C, G), 0)
    onehot = (cls_ids == jnp.broadcast_to(labels, (C, G))).astype(jnp.float32)

    PAD = 16

    def block_step(i, carry):
        vals, idxs = carry
        base = i * B
        pb = preds_ref[pl.ds(base, B), :]
        px1 = pb[:, 0:1]
        py1 = pb[:, 1:2]
        px2 = pb[:, 2:3]
        py2 = pb[:, 3:4]
        area1 = (px2 - px1) * (py2 - py1)
        iw = jnp.maximum(jnp.minimum(px2, gx2) - jnp.maximum(px1, gx1), 0.0)
        ih = jnp.maximum(jnp.minimum(py2, gy2) - jnp.maximum(py1, gy1), 0.0)
        inter = iw * ih
        union = jnp.maximum(area1 + area2 - inter, 1e-6)
        iou = inter / union
        ov_pow = jnp.where(iou > 0.0, jnp.maximum(iou, EPS) ** ALPHA, 0.0)

        cb = cls_ref[pl.ds(base, B), :]
        cls_pow = jax.nn.sigmoid(cb) ** (1.0 - ALPHA)
        cls_sel = jnp.dot(
            cls_pow,
            onehot,
            preferred_element_type=jnp.float32,
            precision=jax.lax.Precision.HIGHEST,
        )
        score = cls_sel * ov_pow

        rid = jax.lax.broadcasted_iota(jnp.int32, (B, G), 0) + base
        pool_v = jnp.concatenate([score, vals], axis=0)
        pool_i = jnp.concatenate([rid, idxs], axis=0)

        vr, ir = [], []
        for _ in range(TOPK):
            m = jnp.max(pool_v, axis=0, keepdims=True)
            ismax = pool_v == m
            sel = jnp.min(
                jnp.where(ismax, pool_i, jnp.int32(2**31 - 1)),
                axis=0,
                keepdims=True,
            )
            vr.append(m)
            ir.append(sel)
            pool_v = jnp.where(pool_i == sel, -1.0, pool_v)

        pad_rows = PAD - TOPK
        vr.append(jnp.full((pad_rows, G), -1.0, jnp.float32))
        ir.append(jnp.full((pad_rows, G), 2**30, jnp.int32))
        return jnp.concatenate(vr, axis=0), jnp.concatenate(ir, axis=0)

    init = (
        jnp.full((PAD, G), -1.0, jnp.float32),
        jnp.full((PAD, G), 2**30, jnp.int32),
    )
    _, idxs = jax.lax.fori_loop(0, NB, block_step, init)
    idx_ref[...] = jnp.concatenate(
        [idxs, jnp.full((PAD, GPAD - G), TRASH, jnp.int32)], axis=1
    )


def _sc_body(
    idx_hbm, nx_hbm, ny_hbm, g1_hbm, g2_hbm, g3_hbm, g4_hbm, bflat_hbm,
    out_hbm,
    idx_v, xa, xb, ya, yb, nx_v, ny_v,
    g1_v, g2_v, g3_v, g4_v, w_v, tmp_i, tmp_w, out_v, w_sh, sem,
):
    n_real = NPAD - 480
    cid = jax.lax.axis_index("c")
    sid = jax.lax.axis_index("s")
    wid = sid * 2 + cid
    per = NPAD // 32
    base = wid * per

    pltpu.sync_copy(idx_hbm.at[pl.ds(0, CAND)], idx_v)

    rot_vecs = [(jax.lax.iota(jnp.int32, 16) + r) & 15 for r in range(1, 16)]

    @pl.when(sid < GPAD // 16)
    def _gauss():
        s = sid * 16
        pltpu.sync_copy(nx_hbm, nx_v)
        pltpu.sync_copy(ny_hbm, ny_v)
        pltpu.sync_copy(g1_hbm, g1_v)
        pltpu.sync_copy(g2_hbm, g2_v)
        pltpu.sync_copy(g3_hbm, g3_v)
        pltpu.sync_copy(g4_hbm, g4_v)

        cps = []
        for k in range(TOPK):
            o = k * GPAD
            iv = jnp.minimum(idx_v[pl.ds(o + s, 16)], n_real - 1) * 4
            for dst, off in ((xa, 0), (ya, 1), (xb, 2), (yb, 3)):
                cp = pltpu.make_async_copy(
                    bflat_hbm.at[iv + off],
                    dst.at[pl.ds(o + s, 16)],
                    sem,
                )
                cp.start()
                cps.append(cp)
        for cp in cps:
            cp.wait()

        glane = jax.lax.iota(jnp.int32, 16)
        laneok = (glane + s) < 100
        c1 = g1_v[pl.ds(s, 16)]
        c2 = g2_v[pl.ds(s, 16)]
        c3 = g3_v[pl.ds(s, 16)]
        c4 = g4_v[pl.ds(s, 16)]
        cxs, cys, nxs, nys = [], [], [], []
        sx = jnp.zeros((16,), jnp.float32)
        sy = jnp.zeros((16,), jnp.float32)
        for k in range(TOPK):
            o = k * GPAD + s
            cx = (xa[pl.ds(o, 16)] + xb[pl.ds(o, 16)]) * 0.5
            cy = (ya[pl.ds(o, 16)] + yb[pl.ds(o, 16)]) * 0.5
            nk = nx_v[pl.ds(o, 16)]
            mk = ny_v[pl.ds(o, 16)]
            cxs.append(cx)
            cys.append(cy)
            nxs.append(nk)
            nys.append(mk)
            sx = sx + (cx + nk)
            sy = sy + (cy + mk)
        mx = sx / 9.0
        my = sy / 9.0
        sxx = jnp.zeros((16,), jnp.float32)
        sxy = jnp.zeros((16,), jnp.float32)
        syy = jnp.zeros((16,), jnp.float32)
        for k in range(TOPK):
            dxn = (cxs[k] + nxs[k]) - mx
            dyn = (cys[k] + nys[k]) - my
            sxx = sxx + dxn * dxn
            sxy = sxy + dxn * dyn
            syy = syy + dyn * dyn
        sxx = sxx / 9.0
        sxy = sxy / 9.0
        syy = syy / 9.0
        den = sxx * syy - sxy * sxy + 1e-10
        i00 = syy / den
        i01 = -sxy / den
        i11 = sxx / den
        for k in range(TOPK):
            px = cxs[k] - mx
            py = cys[k] - my
            q = i00 * px * px + 2.0 * i01 * px * py + i11 * py * py
            w = jnp.exp(-0.5 * q)
            valid = (
                (cxs[k] - c1 > EPS)
                & (cys[k] - c2 > EPS)
                & (c3 - cxs[k] > EPS)
                & (c4 - cys[k] > EPS)
            )
            w = jnp.where(valid & laneok, w, 0.0)
            iv = idx_v[pl.ds(k * GPAD + s, 16)]
            tmp_i[...] = iv
            tmp_w[...] = w
            for rv in rot_vecs:
                ir = plsc.load_gather(tmp_i, [rv])
                wr = plsc.load_gather(tmp_w, [rv])
                w = jnp.where(ir == iv, jnp.maximum(w, wr), w)
            w_v[pl.ds(k * GPAD + s, 16)] = w
            pltpu.sync_copy(
                w_v.at[pl.ds(k * GPAD + s, 16)],
                w_sh.at[pl.ds(k * GPAD + s, 16)],
            )

    plsc.subcore_barrier()
    pltpu.sync_copy(w_sh, w_v)

    zeros16 = jnp.zeros((16,), jnp.float32)

    def zero(i, _):
        out_v[pl.ds(i * 16, 16)] = zeros16
        return 0

    jax.lax.fori_loop(0, per // 16, zero, 0)

    def scat(j, _):
        iv = idx_v[pl.ds(j * 16, 16)]
        wv = w_v[pl.ds(j * 16, 16)]
        msk = (iv >= base) & (iv < base + per)
        loc = jnp.where(msk, iv - base, 0)
        old = plsc.load_gather(out_v, [loc], mask=msk)
        new = jnp.where(msk, jnp.maximum(old, wv), wv)
        plsc.store_scatter(out_v, [loc], new, mask=msk)
        return 0

    jax.lax.fori_loop(0, CAND // 16, scat, 0)
    pltpu.sync_copy(out_v, out_hbm.at[pl.ds(base, per)])


def kernel(bboxes, cls_scores, bbox_preds, gt_bboxes, bbox_levels, gt_labels):
    del bbox_levels
    N, C = cls_scores.shape
    G = gt_bboxes.shape[0]

    gt_t = gt_bboxes[:, :4].T.astype(jnp.float32)
    labels2 = gt_labels.astype(jnp.int32).reshape(1, G)

    idx_o = pl.pallas_call(
        _tc_body,
        out_shape=jax.ShapeDtypeStruct((16, GPAD), jnp.int32),
    )(
        bbox_preds.astype(jnp.float32),
        cls_scores.astype(jnp.float32),
        gt_t,
        labels2,
    )

    noise = (
        jax.random.uniform(jax.random.key(1), (G, TOPK, 2), dtype=jnp.float32) - 0.5
    ) * 0.1
    nxp = jnp.zeros((TOPK, GPAD), jnp.float32).at[:, :G].set(noise[:, :, 0].T)
    nyp = jnp.zeros((TOPK, GPAD), jnp.float32).at[:, :G].set(noise[:, :, 1].T)
    gpad = jnp.zeros((4, GPAD), jnp.float32).at[:, :G].set(gt_t)
    bflat = bboxes[:, :4].astype(jnp.float32).reshape(-1)

    mesh = plsc.VectorSubcoreMesh(core_axis_name="c", subcore_axis_name="s")
    sc = functools.partial(
        pl.kernel,
        mesh=mesh,
        compiler_params=pltpu.CompilerParams(needs_layout_passes=False),
        out_type=jax.ShapeDtypeStruct((NPAD,), jnp.float32),
        scratch_types=[
            pltpu.VMEM((CAND,), jnp.int32),
            pltpu.VMEM((CAND,), jnp.float32),
            pltpu.VMEM((CAND,), jnp.float32),
            pltpu.VMEM((CAND,), jnp.float32),
            pltpu.VMEM((CAND,), jnp.float32),
            pltpu.VMEM((CAND,), jnp.float32),
            pltpu.VMEM((CAND,), jnp.float32),
            pltpu.VMEM((GPAD,), jnp.float32),
            pltpu.VMEM((GPAD,), jnp.float32),
            pltpu.VMEM((GPAD,), jnp.float32),
            pltpu.VMEM((GPAD,), jnp.float32),
            pltpu.VMEM((CAND,), jnp.float32),
            pltpu.VMEM((16,), jnp.int32),
            pltpu.VMEM((16,), jnp.float32),
            pltpu.VMEM((NPAD // 32,), jnp.float32),
            pltpu.VMEM_SHARED((CAND,), jnp.float32),
            pltpu.SemaphoreType.DMA,
        ],
    )(_sc_body)
    out_pad = sc(
        idx_o.reshape(-1),
        nxp.reshape(-1),
        nyp.reshape(-1),
        gpad[0],
        gpad[1],
        gpad[2],
        gpad[3],
        bflat,
    )
    return out_pad[:N]


# speedup vs baseline: 2.2110x; 1.0832x over previous
import functools

import jax
import jax.numpy as jnp
from jax.experimental import pallas as pl
from jax.experimental.pallas import tpu as pltpu
from jax.experimental.pallas import tpu_sc as plsc

EPS = 1e-10
ALPHA = 0.8
TOPK = 9
NPAD = 20480
GPAD = 112
CAND = TOPK * GPAD
TRASH = 20000


def _tc_body(preds_ref, cls_ref, gtt_ref, lab_ref, idx_ref):
    N = preds_ref.shape[0]
    C = cls_ref.shape[1]
    G = gtt_ref.shape[1]
    B = 4000
    NB = N // B

    gx1 = gtt_ref[0:1, :]
    gy1 = gtt_ref[1:2, :]
    gx2 = gtt_ref[2:3, :]
    gy2 = gtt_ref[3:4, :]
    area2 = (gx2 - gx1) * (gy2 - gy1)

    labels = lab_ref[0:1, :]
    cls_ids = jax.lax.broadcasted_iota(jnp.int32, (C, G), 0)
    onehot = (cls_ids == jnp.broadcast_to(labels, (C, G))).astype(jnp.float32)

    PAD = 16

    def block_step(i, carry):
        vals, idxs = carry
        base = i * B
        pb = preds_ref[pl.ds(base, B), :]
        px1 = pb[:, 0:1]
        py1 = pb[:, 1:2]
        px2 = pb[:, 2:3]
        py2 = pb[:, 3:4]
        area1 = (px2 - px1) * (py2 - py1)
        iw = jnp.maximum(jnp.minimum(px2, gx2) - jnp.maximum(px1, gx1), 0.0)
        ih = jnp.maximum(jnp.minimum(py2, gy2) - jnp.maximum(py1, gy1), 0.0)
        inter = iw * ih
        union = jnp.maximum(area1 + area2 - inter, 1e-6)
        iou = inter / union
        ov_pow = jnp.where(iou > 0.0, jnp.maximum(iou, EPS) ** ALPHA, 0.0)

        cb = cls_ref[pl.ds(base, B), :]
        cls_pow = jax.nn.sigmoid(cb) ** (1.0 - ALPHA)
        cls_sel = jnp.dot(
            cls_pow,
            onehot,
            preferred_element_type=jnp.float32,
            precision=jax.lax.Precision.HIGHEST,
        )
        score = cls_sel * ov_pow

        rid = jax.lax.broadcasted_iota(jnp.int32, (B, G), 0) + base
        pool_v = jnp.concatenate([vals, score], axis=0)
        pool_i = jnp.concatenate([idxs, rid], axis=0)

        H = (B + PAD) // 2
        a_v, b_v = pool_v[:H], pool_v[H:]
        a_i, b_i = pool_i[:H], pool_i[H:]
        awins = a_v >= b_v
        w_v = jnp.where(awins, a_v, b_v)
        w_i = jnp.where(awins, a_i, b_i)
        l_v = jnp.where(awins, b_v, a_v)
        l_i = jnp.where(awins, b_i, a_i)

        vr, ir = [], []
        for _ in range(TOPK):
            m = jnp.max(w_v, axis=0, keepdims=True)
            ismax = w_v == m
            sel = jnp.min(
                jnp.where(ismax, w_i, jnp.int32(2**31 - 1)),
                axis=0,
                keepdims=True,
            )
            vr.append(m)
            ir.append(sel)
            issel = w_i == sel
            w_v = jnp.where(issel, l_v, w_v)
            w_i = jnp.where(issel, l_i, w_i)
            l_v = jnp.where(issel, -1.0, l_v)

        pad_rows = PAD - TOPK
        vr.append(jnp.full((pad_rows, G), -1.0, jnp.float32))
        ir.append(jnp.full((pad_rows, G), 2**30, jnp.int32))
        return jnp.concatenate(vr, axis=0), jnp.concatenate(ir, axis=0)

    init = (
        jnp.full((PAD, G), -1.0, jnp.float32),
        jnp.full((PAD, G), 2**30, jnp.int32),
    )
    _, idxs = jax.lax.fori_loop(0, NB, block_step, init)
    idx_ref[...] = jnp.concatenate(
        [idxs, jnp.full((PAD, GPAD - G), TRASH, jnp.int32)], axis=1
    )


def _sc_body(
    idx_hbm, nx_hbm, ny_hbm, g1_hbm, g2_hbm, g3_hbm, g4_hbm, bflat_hbm,
    out_hbm,
    idx_v, xa, xb, ya, yb, nx_v, ny_v,
    g1_v, g2_v, g3_v, g4_v, w_v, tmp_i, tmp_w, out_v, w_sh, sem,
):
    n_real = NPAD - 480
    cid = jax.lax.axis_index("c")
    sid = jax.lax.axis_index("s")
    wid = sid * 2 + cid
    per = NPAD // 32
    base = wid * per

    pltpu.sync_copy(idx_hbm.at[pl.ds(0, CAND)], idx_v)

    rot_vecs = [(jax.lax.iota(jnp.int32, 16) + r) & 15 for r in range(1, 16)]

    @pl.when(sid < GPAD // 16)
    def _gauss():
        s = sid * 16
        pltpu.sync_copy(nx_hbm, nx_v)
        pltpu.sync_copy(ny_hbm, ny_v)
        pltpu.sync_copy(g1_hbm, g1_v)
        pltpu.sync_copy(g2_hbm, g2_v)
        pltpu.sync_copy(g3_hbm, g3_v)
        pltpu.sync_copy(g4_hbm, g4_v)

        cps = []
        for k in range(TOPK):
            o = k * GPAD
            iv = jnp.minimum(idx_v[pl.ds(o + s, 16)], n_real - 1) * 4
            for dst, off in ((xa, 0), (ya, 1), (xb, 2), (yb, 3)):
                cp = pltpu.make_async_copy(
                    bflat_hbm.at[iv + off],
                    dst.at[pl.ds(o + s, 16)],
                    sem,
                )
                cp.start()
                cps.append(cp)
        for cp in cps:
            cp.wait()

        glane = jax.lax.iota(jnp.int32, 16)
        laneok = (glane + s) < 100
        c1 = g1_v[pl.ds(s, 16)]
        c2 = g2_v[pl.ds(s, 16)]
        c3 = g3_v[pl.ds(s, 16)]
        c4 = g4_v[pl.ds(s, 16)]
        cxs, cys, nxs, nys = [], [], [], []
        sx = jnp.zeros((16,), jnp.float32)
        sy = jnp.zeros((16,), jnp.float32)
        for k in range(TOPK):
            o = k * GPAD + s
            cx = (xa[pl.ds(o, 16)] + xb[pl.ds(o, 16)]) * 0.5
            cy = (ya[pl.ds(o, 16)] + yb[pl.ds(o, 16)]) * 0.5
            nk = nx_v[pl.ds(o, 16)]
            mk = ny_v[pl.ds(o, 16)]
            cxs.append(cx)
            cys.append(cy)
            nxs.append(nk)
            nys.append(mk)
            sx = sx + (cx + nk)
            sy = sy + (cy + mk)
        mx = sx / 9.0
        my = sy / 9.0
        sxx = jnp.zeros((16,), jnp.float32)
        sxy = jnp.zeros((16,), jnp.float32)
        syy = jnp.zeros((16,), jnp.float32)
        for k in range(TOPK):
            dxn = (cxs[k] + nxs[k]) - mx
            dyn = (cys[k] + nys[k]) - my
            sxx = sxx + dxn * dxn
            sxy = sxy + dxn * dyn
            syy = syy + dyn * dyn
        sxx = sxx / 9.0
        sxy = sxy / 9.0
        syy = syy / 9.0
        den = sxx * syy - sxy * sxy + 1e-10
        i00 = syy / den
        i01 = -sxy / den
        i11 = sxx / den
        for k in range(TOPK):
            px = cxs[k] - mx
            py = cys[k] - my
            q = i00 * px * px + 2.0 * i01 * px * py + i11 * py * py
            w = jnp.exp(-0.5 * q)
            valid = (
                (cxs[k] - c1 > EPS)
                & (cys[k] - c2 > EPS)
                & (c3 - cxs[k] > EPS)
                & (c4 - cys[k] > EPS)
            )
            w = jnp.where(valid & laneok, w, 0.0)
            iv = idx_v[pl.ds(k * GPAD + s, 16)]
            tmp_i[...] = iv
            tmp_w[...] = w
            for rv in rot_vecs:
                ir = plsc.load_gather(tmp_i, [rv])
                wr = plsc.load_gather(tmp_w, [rv])
                w = jnp.where(ir == iv, jnp.maximum(w, wr), w)
            w_v[pl.ds(k * GPAD + s, 16)] = w
            pltpu.sync_copy(
                w_v.at[pl.ds(k * GPAD + s, 16)],
                w_sh.at[pl.ds(k * GPAD + s, 16)],
            )

    plsc.subcore_barrier()
    pltpu.sync_copy(w_sh, w_v)

    zeros16 = jnp.zeros((16,), jnp.float32)

    def zero(i, _):
        out_v[pl.ds(i * 16, 16)] = zeros16
        return 0

    jax.lax.fori_loop(0, per // 16, zero, 0)

    def scat(j, _):
        iv = idx_v[pl.ds(j * 16, 16)]
        wv = w_v[pl.ds(j * 16, 16)]
        msk = (iv >= base) & (iv < base + per)
        loc = jnp.where(msk, iv - base, 0)
        old = plsc.load_gather(out_v, [loc], mask=msk)
        new = jnp.where(msk, jnp.maximum(old, wv), wv)
        plsc.store_scatter(out_v, [loc], new, mask=msk)
        return 0

    jax.lax.fori_loop(0, CAND // 16, scat, 0)
    pltpu.sync_copy(out_v, out_hbm.at[pl.ds(base, per)])


def kernel(bboxes, cls_scores, bbox_preds, gt_bboxes, bbox_levels, gt_labels):
    del bbox_levels
    N, C = cls_scores.shape
    G = gt_bboxes.shape[0]

    gt_t = gt_bboxes[:, :4].T.astype(jnp.float32)
    labels2 = gt_labels.astype(jnp.int32).reshape(1, G)

    idx_o = pl.pallas_call(
        _tc_body,
        out_shape=jax.ShapeDtypeStruct((16, GPAD), jnp.int32),
    )(
        bbox_preds.astype(jnp.float32),
        cls_scores.astype(jnp.float32),
        gt_t,
        labels2,
    )

    noise = (
        jax.random.uniform(jax.random.key(1), (G, TOPK, 2), dtype=jnp.float32) - 0.5
    ) * 0.1
    nxp = jnp.zeros((TOPK, GPAD), jnp.float32).at[:, :G].set(noise[:, :, 0].T)
    nyp = jnp.zeros((TOPK, GPAD), jnp.float32).at[:, :G].set(noise[:, :, 1].T)
    gpad = jnp.zeros((4, GPAD), jnp.float32).at[:, :G].set(gt_t)
    bflat = bboxes[:, :4].astype(jnp.float32).reshape(-1)

    mesh = plsc.VectorSubcoreMesh(core_axis_name="c", subcore_axis_name="s")
    sc = functools.partial(
        pl.kernel,
        mesh=mesh,
        compiler_params=pltpu.CompilerParams(needs_layout_passes=False),
        out_type=jax.ShapeDtypeStruct((NPAD,), jnp.float32),
        scratch_types=[
            pltpu.VMEM((CAND,), jnp.int32),
            pltpu.VMEM((CAND,), jnp.float32),
            pltpu.VMEM((CAND,), jnp.float32),
            pltpu.VMEM((CAND,), jnp.float32),
            pltpu.VMEM((CAND,), jnp.float32),
            pltpu.VMEM((CAND,), jnp.float32),
            pltpu.VMEM((CAND,), jnp.float32),
            pltpu.VMEM((GPAD,), jnp.float32),
            pltpu.VMEM((GPAD,), jnp.float32),
            pltpu.VMEM((GPAD,), jnp.float32),
            pltpu.VMEM((GPAD,), jnp.float32),
            pltpu.VMEM((CAND,), jnp.float32),
            pltpu.VMEM((16,), jnp.int32),
            pltpu.VMEM((16,), jnp.float32),
            pltpu.VMEM((NPAD // 32,), jnp.float32),
            pltpu.VMEM_SHARED((CAND,), jnp.float32),
            pltpu.SemaphoreType.DMA,
        ],
    )(_sc_body)
    out_pad = sc(
        idx_o.reshape(-1),
        nxp.reshape(-1),
        nyp.reshape(-1),
        gpad[0],
        gpad[1],
        gpad[2],
        gpad[3],
        bflat,
    )
    return out_pad[:N]
